# Initial kernel scaffold; baseline (speedup 1.0000x reference)
import functools

import jax
import jax.numpy as jnp
from jax import lax
from jax.experimental import pallas as pl
from jax.experimental.pallas import tpu as pltpu
from jax.experimental.pallas import tpu_sc as plsc

N = 10000
E = 320000
D = 128
T = 10
NC = 2
NS = 16
NW = NC * NS
EPW = E // NW
CH = 80
NCH = EPW // CH
RPT = N // NS
ZR = 125

_SC_MESH = plsc.VectorSubcoreMesh(core_axis_name="c", subcore_axis_name="s")


def _zero_vmem_f32(ref, rows, cols):
    zeros16 = jnp.zeros((16,), jnp.float32)

    def zrow(r, _):
        for k in range(cols // 16):
            ref[r, pl.ds(k * 16, 16)] = zeros16
        return 0

    lax.fori_loop(0, rows, zrow, 0)


@functools.partial(
    pl.kernel,
    out_type=jax.ShapeDtypeStruct((NC, N, D), jnp.float32),
    mesh=_SC_MESH,
    scratch_types=[
        pltpu.VMEM((EPW,), jnp.int32),
        pltpu.VMEM((NCH, CH), jnp.int32),
        pltpu.VMEM((CH, D), jnp.float32),
        pltpu.VMEM((ZR, D), jnp.float32),
        pltpu.SemaphoreType.DMA,
        pltpu.VMEM_SHARED((N, D), jnp.float32),
    ],
)
def _segsum_sc(ei_hbm, h_hbm, out_hbm, src_v, dst_v, rows_v, zbuf_v, sem, acc):
    c = lax.axis_index("c")
    s = lax.axis_index("s")
    wid = s * NC + c
    base = wid * EPW

    _zero_vmem_f32(zbuf_v, ZR, D)

    def zblk(b, _):
        pltpu.sync_copy(zbuf_v, acc.at[pl.ds(s * RPT + b * ZR, ZR)])
        return 0

    lax.fori_loop(0, RPT // ZR, zblk, 0)

    pltpu.sync_copy(ei_hbm.at[0, pl.ds(base, EPW)], src_v)

    def ldst(j, _):
        pltpu.sync_copy(ei_hbm.at[1, pl.ds(base + j * CH, CH)], dst_v.at[j])
        return 0

    lax.fori_loop(0, NCH, ldst, 0)

    plsc.subcore_barrier()

    def chunk(j, _):
        pltpu.async_copy(h_hbm.at[src_v.at[pl.ds(j * CH, CH)]], rows_v, sem).wait()
        pltpu.sync_copy(rows_v, acc.at[dst_v.at[j]], add=True)
        return 0

    lax.fori_loop(0, NCH, chunk, 0)

    plsc.subcore_barrier()

    pltpu.sync_copy(acc.at[pl.ds(s * RPT, RPT)], out_hbm.at[c, pl.ds(s * RPT, RPT)])


@functools.partial(
    pl.kernel,
    out_type=jax.ShapeDtypeStruct((NW, N * T), jnp.float32),
    mesh=_SC_MESH,
    scratch_types=[
        pltpu.VMEM((N * T,), jnp.float32),
        pltpu.VMEM((NCH, CH), jnp.int32),
        pltpu.VMEM((NCH, CH), jnp.int32),
    ],
)
def _counts_sc(ei_hbm, et_hbm, out_hbm, cnt_v, dst_v, typ_v):
    c = lax.axis_index("c")
    s = lax.axis_index("s")
    wid = s * NC + c
    base = wid * EPW

    zeros16 = jnp.zeros((16,), jnp.float32)

    def zrow(r, _):
        for k in range(8):
            cnt_v[pl.ds(r * 128 + k * 16, 16)] = zeros16
        return 0

    lax.fori_loop(0, (N * T) // 128, zrow, 0)

    def ldst(j, _):
        pltpu.sync_copy(ei_hbm.at[1, pl.ds(base + j * CH, CH)], dst_v.at[j])
        pltpu.sync_copy(et_hbm.at[pl.ds(base + j * CH, CH)], typ_v.at[j])
        return 0

    lax.fori_loop(0, NCH, ldst, 0)

    ones16 = jnp.ones((16,), jnp.float32)

    def chunk(j, _):
        def grp(k, _):
            d = dst_v[j, pl.ds(k * 16, 16)]
            t = typ_v[j, pl.ds(k * 16, 16)]
            plsc.addupdate_scatter(cnt_v, [d * T + t], ones16)
            return 0

        lax.fori_loop(0, CH // 16, grp, 0)
        return 0

    lax.fori_loop(0, NCH, chunk, 0)

    pltpu.sync_copy(cnt_v, out_hbm.at[wid])


R = 1000
GRID = N // R


def _mlp1_tc_body(x_ref, p_ref, c_ref, et_ref, wa_ref, ba_ref, wb_ref, bb_ref,
                  h_ref, eb_ref):
    cs = jnp.sum(c_ref[...], axis=0)
    eb = jnp.dot(cs, et_ref[...], preferred_element_type=jnp.float32)
    z = 2.0 * x_ref[...] + p_ref[0] + p_ref[1] + eb
    a = jnp.dot(z, wa_ref[...], preferred_element_type=jnp.float32) + ba_ref[...]
    a = jnp.where(a > 0, a, 0.01 * a)
    h_ref[...] = jnp.dot(a, wb_ref[...], preferred_element_type=jnp.float32) + bb_ref[...]
    eb_ref[...] = eb


def _mlp_tc_body(x_ref, p_ref, eb_ref, wa_ref, ba_ref, wb_ref, bb_ref, h_ref):
    z = 2.0 * x_ref[...] + p_ref[0] + p_ref[1] + eb_ref[...]
    a = jnp.dot(z, wa_ref[...], preferred_element_type=jnp.float32) + ba_ref[...]
    a = jnp.where(a > 0, a, 0.01 * a)
    h_ref[...] = jnp.dot(a, wb_ref[...], preferred_element_type=jnp.float32) + bb_ref[...]


_mlp1_tc = pl.pallas_call(
    _mlp1_tc_body,
    grid=(GRID,),
    in_specs=[
        pl.BlockSpec((R, D), lambda i: (i, 0)),
        pl.BlockSpec((NC, R, D), lambda i: (0, i, 0)),
        pl.BlockSpec((NW, R, T), lambda i: (0, i, 0)),
        pl.BlockSpec((T, D), lambda i: (0, 0)),
        pl.BlockSpec((D, D), lambda i: (0, 0)),
        pl.BlockSpec((1, D), lambda i: (0, 0)),
        pl.BlockSpec((D, D), lambda i: (0, 0)),
        pl.BlockSpec((1, D), lambda i: (0, 0)),
    ],
    out_specs=[
        pl.BlockSpec((R, D), lambda i: (i, 0)),
        pl.BlockSpec((R, D), lambda i: (i, 0)),
    ],
    out_shape=[
        jax.ShapeDtypeStruct((N, D), jnp.float32),
        jax.ShapeDtypeStruct((N, D), jnp.float32),
    ],
)

_mlp_tc = pl.pallas_call(
    _mlp_tc_body,
    grid=(GRID,),
    in_specs=[
        pl.BlockSpec((R, D), lambda i: (i, 0)),
        pl.BlockSpec((NC, R, D), lambda i: (0, i, 0)),
        pl.BlockSpec((R, D), lambda i: (i, 0)),
        pl.BlockSpec((D, D), lambda i: (0, 0)),
        pl.BlockSpec((1, D), lambda i: (0, 0)),
        pl.BlockSpec((D, D), lambda i: (0, 0)),
        pl.BlockSpec((1, D), lambda i: (0, 0)),
    ],
    out_specs=pl.BlockSpec((R, D), lambda i: (i, 0)),
    out_shape=jax.ShapeDtypeStruct((N, D), jnp.float32),
)


def kernel(x, edge_index, edge_type, e_table,
           W1a, b1a, W1b, b1b, W2a, b2a, W2b, b2b, W3a, b3a, W3b, b3b):
    b1a_, b1b_ = b1a.reshape(1, D), b1b.reshape(1, D)
    b2a_, b2b_ = b2a.reshape(1, D), b2b.reshape(1, D)
    b3a_, b3b_ = b3a.reshape(1, D), b3b.reshape(1, D)

    C = _counts_sc(edge_index, edge_type).reshape(NW, N, T)
    P0 = _segsum_sc(edge_index, x)
    h1, eb = _mlp1_tc(x, P0, C, e_table, W1a, b1a_, W1b, b1b_)
    P1 = _segsum_sc(edge_index, h1)
    h2 = _mlp_tc(h1, P1, eb, W2a, b2a_, W2b, b2b_)
    P2 = _segsum_sc(edge_index, h2)
    h3 = _mlp_tc(h2, P2, eb, W3a, b3a_, W3b, b3b_)
    return h3


# speedup vs baseline: 5.7613x; 5.7613x over previous
import functools

import jax
import jax.numpy as jnp
from jax import lax
from jax.experimental import pallas as pl
from jax.experimental.pallas import tpu as pltpu
from jax.experimental.pallas import tpu_sc as plsc

N = 10000
E = 320000
D = 128
T = 10
NC = 2
NS = 16
NW = NC * NS
EPW = E // NW
CH = 80
NCH = EPW // CH
NP_ = 10240
RPT = NP_ // NS

_SC_MESH = plsc.VectorSubcoreMesh(core_axis_name="c", subcore_axis_name="s")


def _zero_vmem_f32(ref, rows, cols):
    zeros16 = jnp.zeros((16,), jnp.float32)

    def zrow(r, _):
        for k in range(cols // 16):
            ref[r, pl.ds(k * 16, 16)] = zeros16
        return 0

    lax.fori_loop(0, rows, zrow, 0)


@functools.partial(
    pl.kernel,
    out_type=jax.ShapeDtypeStruct((NC, NP_, D), jnp.float32),
    mesh=_SC_MESH,
    scratch_types=[
        pltpu.VMEM((EPW,), jnp.int32),
        pltpu.VMEM((NCH, CH), jnp.int32),
        pltpu.VMEM((CH, D), jnp.float32),
        pltpu.SemaphoreType.DMA,
        pltpu.VMEM_SHARED((NP_, D), jnp.float32),
    ],
)
def _segsum_sc(src_hbm, dst_hbm, h_hbm, out_hbm, src_v, dst_v, rows_v, sem, acc):
    c = lax.axis_index("c")
    s = lax.axis_index("s")
    wid = s * NC + c
    base = wid * EPW

    _zero_vmem_f32(rows_v, CH, D)

    def zblk(b, _):
        pltpu.sync_copy(rows_v, acc.at[pl.ds(s * RPT + b * CH, CH)])
        return 0

    lax.fori_loop(0, RPT // CH, zblk, 0)

    pltpu.sync_copy(src_hbm.at[pl.ds(base, EPW)], src_v)

    def ldst(j, _):
        pltpu.sync_copy(dst_hbm.at[pl.ds(base + j * CH, CH)], dst_v.at[j])
        return 0

    lax.fori_loop(0, NCH, ldst, 0)

    plsc.subcore_barrier()

    def chunk(j, _):
        pltpu.async_copy(h_hbm.at[src_v.at[pl.ds(j * CH, CH)]], rows_v, sem).wait()
        pltpu.sync_copy(rows_v, acc.at[dst_v.at[j]], add=True)
        return 0

    lax.fori_loop(0, NCH, chunk, 0)

    plsc.subcore_barrier()

    pltpu.sync_copy(acc.at[pl.ds(s * RPT, RPT)], out_hbm.at[c, pl.ds(s * RPT, RPT)])


@functools.partial(
    pl.kernel,
    out_type=jax.ShapeDtypeStruct((NW, T * NP_), jnp.float32),
    mesh=_SC_MESH,
    compiler_params=pltpu.CompilerParams(needs_layout_passes=False),
    scratch_types=[
        pltpu.VMEM((T * NP_,), jnp.float32),
        pltpu.VMEM((CH,), jnp.int32),
        pltpu.VMEM((CH,), jnp.int32),
    ],
)
def _counts_sc(dst_hbm, et_hbm, out_hbm, cnt_v, dst_v, typ_v):
    c = lax.axis_index("c")
    s = lax.axis_index("s")
    wid = s * NC + c
    base = wid * EPW

    zeros16 = jnp.zeros((16,), jnp.float32)

    def zrow(r, _):
        for k in range(8):
            cnt_v[pl.ds(r * 128 + k * 16, 16)] = zeros16
        return 0

    lax.fori_loop(0, (T * NP_) // 128, zrow, 0)

    ones16 = jnp.ones((16,), jnp.float32)

    def chunk(j, _):
        pltpu.sync_copy(dst_hbm.at[pl.ds(base + j * CH, CH)], dst_v)
        pltpu.sync_copy(et_hbm.at[pl.ds(base + j * CH, CH)], typ_v)

        def grp(k, _):
            d = dst_v[pl.ds(k * 16, 16)]
            t = typ_v[pl.ds(k * 16, 16)]
            plsc.addupdate_scatter(cnt_v, [t * NP_ + d], ones16)
            return 0

        lax.fori_loop(0, CH // 16, grp, 0)
        return 0

    lax.fori_loop(0, NCH, chunk, 0)

    pltpu.sync_copy(cnt_v, out_hbm.at[wid])


R = 1000
GRID = N // R


def _eb_tc_body(c_ref, et_ref, eb_ref):
    cs = jnp.sum(c_ref[...], axis=0)
    eb = jax.lax.dot_general(cs, et_ref[...], ((( 0,), (0,)), ((), ())),
                             preferred_element_type=jnp.float32)
    eb_ref[...] = eb[:N]


_eb_tc = pl.pallas_call(
    _eb_tc_body,
    in_specs=[
        pl.BlockSpec((NW, T, NP_), lambda: (0, 0, 0)),
        pl.BlockSpec((T, D), lambda: (0, 0)),
    ],
    out_specs=pl.BlockSpec((N, D), lambda: (0, 0)),
    out_shape=jax.ShapeDtypeStruct((N, D), jnp.float32),
)


def _mlp_tc_body(x_ref, p_ref, eb_ref, wa_ref, ba_ref, wb_ref, bb_ref, h_ref):
    z = 2.0 * x_ref[...] + p_ref[0] + p_ref[1] + eb_ref[...]
    a = jnp.dot(z, wa_ref[...], preferred_element_type=jnp.float32) + ba_ref[...]
    a = jnp.where(a > 0, a, 0.01 * a)
    h_ref[...] = jnp.dot(a, wb_ref[...], preferred_element_type=jnp.float32) + bb_ref[...]


_mlp_tc = pl.pallas_call(
    _mlp_tc_body,
    grid=(GRID,),
    in_specs=[
        pl.BlockSpec((R, D), lambda i: (i, 0)),
        pl.BlockSpec((NC, R, D), lambda i: (0, i, 0)),
        pl.BlockSpec((R, D), lambda i: (i, 0)),
        pl.BlockSpec((D, D), lambda i: (0, 0)),
        pl.BlockSpec((1, D), lambda i: (0, 0)),
        pl.BlockSpec((D, D), lambda i: (0, 0)),
        pl.BlockSpec((1, D), lambda i: (0, 0)),
    ],
    out_specs=pl.BlockSpec((R, D), lambda i: (i, 0)),
    out_shape=jax.ShapeDtypeStruct((N, D), jnp.float32),
)


def kernel(x, edge_index, edge_type, e_table,
           W1a, b1a, W1b, b1b, W2a, b2a, W2b, b2b, W3a, b3a, W3b, b3b):
    b1a_, b1b_ = b1a.reshape(1, D), b1b.reshape(1, D)
    b2a_, b2b_ = b2a.reshape(1, D), b2b.reshape(1, D)
    b3a_, b3b_ = b3a.reshape(1, D), b3b.reshape(1, D)

    src = edge_index[0]
    dst = edge_index[1]
    C = _counts_sc(dst, edge_type).reshape(NW, T, NP_)
    eb = _eb_tc(C, e_table)
    P0 = _segsum_sc(src, dst, x)[:, :N]
    h1 = _mlp_tc(x, P0, eb, W1a, b1a_, W1b, b1b_)
    P1 = _segsum_sc(src, dst, h1)[:, :N]
    h2 = _mlp_tc(h1, P1, eb, W2a, b2a_, W2b, b2b_)
    P2 = _segsum_sc(src, dst, h2)[:, :N]
    h3 = _mlp_tc(h2, P2, eb, W3a, b3a_, W3b, b3b_)
    return h3


# speedup vs baseline: 9.7630x; 1.6946x over previous
import functools

import jax
import jax.numpy as jnp
from jax import lax
from jax.experimental import pallas as pl
from jax.experimental.pallas import tpu as pltpu
from jax.experimental.pallas import tpu_sc as plsc

N = 10000
E = 320000
D = 128
T = 10
NC = 2
NS = 16
NW = NC * NS
EPW = E // NW
CH = 40
NCH = EPW // CH
NB = 5
NR = NCH // NB
CCH = 80
NP_ = 10240
RPT = NP_ // NS

_SC_MESH = plsc.VectorSubcoreMesh(core_axis_name="c", subcore_axis_name="s")


def _zero_vmem_f32(ref, rows, cols):
    zeros16 = jnp.zeros((16,), jnp.float32)

    def zrow(r, _):
        for k in range(cols // 16):
            ref[r, pl.ds(k * 16, 16)] = zeros16
        return 0

    lax.fori_loop(0, rows, zrow, 0)


@functools.partial(
    pl.kernel,
    out_type=jax.ShapeDtypeStruct((NC, NP_, D), jnp.float32),
    mesh=_SC_MESH,
    scratch_types=[
        pltpu.VMEM((2, NB, CH), jnp.int32),
        pltpu.VMEM((2, NB, CH), jnp.int32),
        pltpu.VMEM((NB, CH, D), jnp.float32),
    ]
    + [pltpu.SemaphoreType.DMA] * (3 * NB)
    + [pltpu.VMEM_SHARED((NP_, D), jnp.float32)],
)
def _segsum_sc(src_hbm, dst_hbm, h_hbm, out_hbm, sidx, didx, rows, *rest):
    gsem = rest[0:NB]
    ssem = rest[NB:2 * NB]
    isem = rest[2 * NB:3 * NB]
    acc = rest[3 * NB]

    c = lax.axis_index("c")
    s = lax.axis_index("s")
    wid = s * NC + c
    base = wid * EPW

    _zero_vmem_f32(rows.at[0], CH, D)

    def zblk(b, _):
        pltpu.sync_copy(rows.at[0], acc.at[pl.ds(s * RPT + b * CH, CH)])
        return 0

    lax.fori_loop(0, RPT // CH, zblk, 0)

    for b in range(NB):
        pltpu.sync_copy(src_hbm.at[pl.ds(base + b * CH, CH)], sidx.at[0, b])
        pltpu.sync_copy(dst_hbm.at[pl.ds(base + b * CH, CH)], didx.at[0, b])

    plsc.subcore_barrier()

    for b in range(NB):
        pltpu.async_copy(h_hbm.at[sidx.at[0, b]], rows.at[b], gsem[b])

    def round_body(g, _):
        p = lax.rem(g, 2)
        q = 1 - p

        @pl.when(g < NR - 1)
        def _():
            for b in range(NB):
                off = base + ((g + 1) * NB + b) * CH
                pltpu.async_copy(src_hbm.at[pl.ds(off, CH)], sidx.at[q, b], isem[b])
                pltpu.async_copy(dst_hbm.at[pl.ds(off, CH)], didx.at[q, b], isem[b])

        for b in range(NB):
            pltpu.make_async_copy(h_hbm.at[sidx.at[p, b]], rows.at[b], gsem[b]).wait()
            pltpu.async_copy(rows.at[b], acc.at[didx.at[p, b]], ssem[b], add=True)

        for b in range(NB):
            pltpu.make_async_copy(rows.at[b], acc.at[didx.at[p, b]], ssem[b]).wait()

            @pl.when(g < NR - 1)
            def _():
                pltpu.make_async_copy(src_hbm.at[pl.ds(base, CH)], sidx.at[q, b], isem[b]).wait()
                pltpu.make_async_copy(dst_hbm.at[pl.ds(base, CH)], didx.at[q, b], isem[b]).wait()
                pltpu.async_copy(h_hbm.at[sidx.at[q, b]], rows.at[b], gsem[b])

        return 0

    lax.fori_loop(0, NR, round_body, 0)

    plsc.subcore_barrier()

    pltpu.sync_copy(acc.at[pl.ds(s * RPT, RPT)], out_hbm.at[c, pl.ds(s * RPT, RPT)])


@functools.partial(
    pl.kernel,
    out_type=jax.ShapeDtypeStruct((NW, T * NP_), jnp.float32),
    mesh=_SC_MESH,
    compiler_params=pltpu.CompilerParams(needs_layout_passes=False),
    scratch_types=[
        pltpu.VMEM((T * NP_,), jnp.float32),
        pltpu.VMEM((CCH,), jnp.int32),
        pltpu.VMEM((CCH,), jnp.int32),
    ],
)
def _counts_sc(dst_hbm, et_hbm, out_hbm, cnt_v, dst_v, typ_v):
    c = lax.axis_index("c")
    s = lax.axis_index("s")
    wid = s * NC + c
    base = wid * EPW

    zeros16 = jnp.zeros((16,), jnp.float32)

    def zrow(r, _):
        for k in range(8):
            cnt_v[pl.ds(r * 128 + k * 16, 16)] = zeros16
        return 0

    lax.fori_loop(0, (T * NP_) // 128, zrow, 0)

    ones16 = jnp.ones((16,), jnp.float32)

    def chunk(j, _):
        pltpu.sync_copy(dst_hbm.at[pl.ds(base + j * CCH, CCH)], dst_v)
        pltpu.sync_copy(et_hbm.at[pl.ds(base + j * CCH, CCH)], typ_v)

        def grp(k, _):
            d = dst_v[pl.ds(k * 16, 16)]
            t = typ_v[pl.ds(k * 16, 16)]
            plsc.addupdate_scatter(cnt_v, [t * NP_ + d], ones16)
            return 0

        lax.fori_loop(0, CCH // 16, grp, 0)
        return 0

    lax.fori_loop(0, EPW // CCH, chunk, 0)

    pltpu.sync_copy(cnt_v, out_hbm.at[wid])


R = 1000
GRID = N // R


def _eb_tc_body(c_ref, et_ref, eb_ref):
    cs = jnp.sum(c_ref[...], axis=0)
    eb = jax.lax.dot_general(cs, et_ref[...], ((( 0,), (0,)), ((), ())),
                             preferred_element_type=jnp.float32)
    eb_ref[...] = eb[:N]


_eb_tc = pl.pallas_call(
    _eb_tc_body,
    in_specs=[
        pl.BlockSpec((NW, T, NP_), lambda: (0, 0, 0)),
        pl.BlockSpec((T, D), lambda: (0, 0)),
    ],
    out_specs=pl.BlockSpec((N, D), lambda: (0, 0)),
    out_shape=jax.ShapeDtypeStruct((N, D), jnp.float32),
)


def _mlp_tc_body(x_ref, p_ref, eb_ref, wa_ref, ba_ref, wb_ref, bb_ref, h_ref):
    z = 2.0 * x_ref[...] + p_ref[0] + p_ref[1] + eb_ref[...]
    a = jnp.dot(z, wa_ref[...], preferred_element_type=jnp.float32) + ba_ref[...]
    a = jnp.where(a > 0, a, 0.01 * a)
    h_ref[...] = jnp.dot(a, wb_ref[...], preferred_element_type=jnp.float32) + bb_ref[...]


_mlp_tc = pl.pallas_call(
    _mlp_tc_body,
    grid=(GRID,),
    in_specs=[
        pl.BlockSpec((R, D), lambda i: (i, 0)),
        pl.BlockSpec((NC, R, D), lambda i: (0, i, 0)),
        pl.BlockSpec((R, D), lambda i: (i, 0)),
        pl.BlockSpec((D, D), lambda i: (0, 0)),
        pl.BlockSpec((1, D), lambda i: (0, 0)),
        pl.BlockSpec((D, D), lambda i: (0, 0)),
        pl.BlockSpec((1, D), lambda i: (0, 0)),
    ],
    out_specs=pl.BlockSpec((R, D), lambda i: (i, 0)),
    out_shape=jax.ShapeDtypeStruct((N, D), jnp.float32),
)


def kernel(x, edge_index, edge_type, e_table,
           W1a, b1a, W1b, b1b, W2a, b2a, W2b, b2b, W3a, b3a, W3b, b3b):
    b1a_, b1b_ = b1a.reshape(1, D), b1b.reshape(1, D)
    b2a_, b2b_ = b2a.reshape(1, D), b2b.reshape(1, D)
    b3a_, b3b_ = b3a.reshape(1, D), b3b.reshape(1, D)

    src = edge_index[0]
    dst = edge_index[1]
    C = _counts_sc(dst, edge_type).reshape(NW, T, NP_)
    eb = _eb_tc(C, e_table)
    P0 = _segsum_sc(src, dst, x)[:, :N]
    h1 = _mlp_tc(x, P0, eb, W1a, b1a_, W1b, b1b_)
    P1 = _segsum_sc(src, dst, h1)[:, :N]
    h2 = _mlp_tc(h1, P1, eb, W2a, b2a_, W2b, b2b_)
    P2 = _segsum_sc(src, dst, h2)[:, :N]
    h3 = _mlp_tc(h2, P2, eb, W3a, b3a_, W3b, b3b_)
    return h3


# speedup vs baseline: 11.5434x; 1.1824x over previous
import functools

import jax
import jax.numpy as jnp
from jax import lax
from jax.experimental import pallas as pl
from jax.experimental.pallas import tpu as pltpu
from jax.experimental.pallas import tpu_sc as plsc

N = 10000
E = 320000
D = 128
T = 10
NC = 2
NS = 16
NW = NC * NS
EPW = E // NW
CH = 40
NCH = EPW // CH
NB = 5
NR = NCH // NB
SCH = 2000
NSCH = EPW // SCH
NP_ = 10240
RPT = NP_ // NS

_SC_MESH = plsc.VectorSubcoreMesh(core_axis_name="c", subcore_axis_name="s")


def _zero_vmem_f32(ref, rows, cols):
    zeros16 = jnp.zeros((16,), jnp.float32)

    def zrow(r, _):
        for k in range(cols // 16):
            ref[r, pl.ds(k * 16, 16)] = zeros16
        return 0

    lax.fori_loop(0, rows, zrow, 0)


@functools.partial(
    pl.kernel,
    out_type=jax.ShapeDtypeStruct((NC, NP_, D), jnp.float32),
    mesh=_SC_MESH,
    scratch_types=[
        pltpu.VMEM((2, NB, CH), jnp.int32),
        pltpu.VMEM((2, NB, CH), jnp.int32),
        pltpu.VMEM((NB, CH, D), jnp.float32),
    ]
    + [pltpu.SemaphoreType.DMA] * (3 * NB)
    + [pltpu.VMEM_SHARED((NP_, D), jnp.float32)],
)
def _segsum_sc(src_hbm, dst_hbm, h_hbm, out_hbm, sidx, didx, rows, *rest):
    gsem = rest[0:NB]
    ssem = rest[NB:2 * NB]
    isem = rest[2 * NB:3 * NB]
    acc = rest[3 * NB]

    c = lax.axis_index("c")
    s = lax.axis_index("s")
    wid = s * NC + c
    base = wid * EPW

    _zero_vmem_f32(rows.at[0], CH, D)

    def zblk(b, _):
        pltpu.sync_copy(rows.at[0], acc.at[pl.ds(s * RPT + b * CH, CH)])
        return 0

    lax.fori_loop(0, RPT // CH, zblk, 0)

    for b in range(NB):
        pltpu.sync_copy(src_hbm.at[pl.ds(base + b * CH, CH)], sidx.at[0, b])
        pltpu.sync_copy(dst_hbm.at[pl.ds(base + b * CH, CH)], didx.at[0, b])

    plsc.subcore_barrier()

    for b in range(NB):
        pltpu.async_copy(h_hbm.at[sidx.at[0, b]], rows.at[b], gsem[b])

    def round_body(g, _):
        p = lax.rem(g, 2)
        q = 1 - p

        @pl.when(g < NR - 1)
        def _():
            for b in range(NB):
                off = base + ((g + 1) * NB + b) * CH
                pltpu.async_copy(src_hbm.at[pl.ds(off, CH)], sidx.at[q, b], isem[b])
                pltpu.async_copy(dst_hbm.at[pl.ds(off, CH)], didx.at[q, b], isem[b])

        for b in range(NB):
            pltpu.make_async_copy(h_hbm.at[sidx.at[p, b]], rows.at[b], gsem[b]).wait()
            pltpu.async_copy(rows.at[b], acc.at[didx.at[p, b]], ssem[b], add=True)

        for b in range(NB):
            pltpu.make_async_copy(rows.at[b], acc.at[didx.at[p, b]], ssem[b]).wait()

            @pl.when(g < NR - 1)
            def _():
                pltpu.make_async_copy(src_hbm.at[pl.ds(base, CH)], sidx.at[q, b], isem[b]).wait()
                pltpu.make_async_copy(dst_hbm.at[pl.ds(base, CH)], didx.at[q, b], isem[b]).wait()
                pltpu.async_copy(h_hbm.at[sidx.at[q, b]], rows.at[b], gsem[b])

        return 0

    lax.fori_loop(0, NR, round_body, 0)

    plsc.subcore_barrier()

    pltpu.sync_copy(acc.at[pl.ds(s * RPT, RPT)], out_hbm.at[c, pl.ds(s * RPT, RPT)])


@functools.partial(
    pl.kernel,
    out_type=jax.ShapeDtypeStruct((NW, T * NP_), jnp.float32),
    mesh=_SC_MESH,
    compiler_params=pltpu.CompilerParams(needs_layout_passes=False),
    scratch_types=[
        pltpu.VMEM((T * NP_,), jnp.float32),
        pltpu.VMEM((SCH,), jnp.int32),
        pltpu.VMEM((SCH,), jnp.int32),
        pltpu.VMEM((SCH,), jnp.int32),
        pltpu.VMEM((SCH,), jnp.int32),
        pltpu.SemaphoreType.DMA,
        pltpu.SemaphoreType.DMA,
    ],
)
def _counts_sc(dst_hbm, et_hbm, out_hbm, cnt_v, dst_v0, dst_v1, typ_v0, typ_v1,
               isem0, isem1):
    c = lax.axis_index("c")
    s = lax.axis_index("s")
    wid = s * NC + c
    base = wid * EPW
    dbuf = (dst_v0, dst_v1)
    tbuf = (typ_v0, typ_v1)
    isem = (isem0, isem1)

    zeros16 = jnp.zeros((16,), jnp.float32)

    def zrow(r, _):
        for k in range(8):
            cnt_v[pl.ds(r * 128 + k * 16, 16)] = zeros16
        return 0

    lax.fori_loop(0, (T * NP_) // 128, zrow, 0)

    pltpu.sync_copy(dst_hbm.at[pl.ds(base, SCH)], dbuf[0])
    pltpu.sync_copy(et_hbm.at[pl.ds(base, SCH)], tbuf[0])

    ones16 = jnp.ones((16,), jnp.float32)

    for j in range(NSCH):
        p = j % 2
        q = 1 - p
        if j < NSCH - 1:
            off = base + (j + 1) * SCH
            pltpu.async_copy(dst_hbm.at[pl.ds(off, SCH)], dbuf[q], isem[0])
            pltpu.async_copy(et_hbm.at[pl.ds(off, SCH)], tbuf[q], isem[1])

        def grp(k, _, p=p):
            d = dbuf[p][pl.ds(k * 16, 16)]
            t = tbuf[p][pl.ds(k * 16, 16)]
            plsc.addupdate_scatter(cnt_v, [t * NP_ + d], ones16)
            return 0

        lax.fori_loop(0, SCH // 16, grp, 0)

        if j < NSCH - 1:
            pltpu.make_async_copy(dst_hbm.at[pl.ds(base, SCH)], dbuf[q], isem[0]).wait()
            pltpu.make_async_copy(et_hbm.at[pl.ds(base, SCH)], tbuf[q], isem[1]).wait()

        pltpu.sync_copy(cnt_v, out_hbm.at[wid])


R = 1000
GRID = N // R


def _eb_tc_body(c_ref, et_ref, eb_ref):
    cs = jnp.sum(c_ref[...], axis=0)
    eb = jax.lax.dot_general(cs, et_ref[...], ((( 0,), (0,)), ((), ())),
                             preferred_element_type=jnp.float32)
    eb_ref[...] = eb[:N]


_eb_tc = pl.pallas_call(
    _eb_tc_body,
    in_specs=[
        pl.BlockSpec((NW, T, NP_), lambda: (0, 0, 0)),
        pl.BlockSpec((T, D), lambda: (0, 0)),
    ],
    out_specs=pl.BlockSpec((N, D), lambda: (0, 0)),
    out_shape=jax.ShapeDtypeStruct((N, D), jnp.float32),
)


def _mlp_tc_body(x_ref, p_ref, eb_ref, wa_ref, ba_ref, wb_ref, bb_ref, h_ref):
    z = 2.0 * x_ref[...] + p_ref[0] + p_ref[1] + eb_ref[...]
    a = jnp.dot(z, wa_ref[...], preferred_element_type=jnp.float32) + ba_ref[...]
    a = jnp.where(a > 0, a, 0.01 * a)
    h_ref[...] = jnp.dot(a, wb_ref[...], preferred_element_type=jnp.float32) + bb_ref[...]


_mlp_tc = pl.pallas_call(
    _mlp_tc_body,
    grid=(GRID,),
    in_specs=[
        pl.BlockSpec((R, D), lambda i: (i, 0)),
        pl.BlockSpec((NC, R, D), lambda i: (0, i, 0)),
        pl.BlockSpec((R, D), lambda i: (i, 0)),
        pl.BlockSpec((D, D), lambda i: (0, 0)),
        pl.BlockSpec((1, D), lambda i: (0, 0)),
        pl.BlockSpec((D, D), lambda i: (0, 0)),
        pl.BlockSpec((1, D), lambda i: (0, 0)),
    ],
    out_specs=pl.BlockSpec((R, D), lambda i: (i, 0)),
    out_shape=jax.ShapeDtypeStruct((N, D), jnp.float32),
)


def kernel(x, edge_index, edge_type, e_table,
           W1a, b1a, W1b, b1b, W2a, b2a, W2b, b2b, W3a, b3a, W3b, b3b):
    b1a_, b1b_ = b1a.reshape(1, D), b1b.reshape(1, D)
    b2a_, b2b_ = b2a.reshape(1, D), b2b.reshape(1, D)
    b3a_, b3b_ = b3a.reshape(1, D), b3b.reshape(1, D)

    src = edge_index[0]
    dst = edge_index[1]
    C = _counts_sc(dst, edge_type).reshape(NW, T, NP_)
    eb = _eb_tc(C, e_table)
    P0 = _segsum_sc(src, dst, x)[:, :N]
    h1 = _mlp_tc(x, P0, eb, W1a, b1a_, W1b, b1b_)
    P1 = _segsum_sc(src, dst, h1)[:, :N]
    h2 = _mlp_tc(h1, P1, eb, W2a, b2a_, W2b, b2b_)
    P2 = _segsum_sc(src, dst, h2)[:, :N]
    h3 = _mlp_tc(h2, P2, eb, W3a, b3a_, W3b, b3b_)
    return h3


# speedup vs baseline: 12.0263x; 1.0418x over previous
import functools

import jax
import jax.numpy as jnp
from jax import lax
from jax.experimental import pallas as pl
from jax.experimental.pallas import tpu as pltpu
from jax.experimental.pallas import tpu_sc as plsc

N = 10000
E = 320000
D = 128
T = 10
NC = 2
NS = 16
NW = NC * NS
EPW = E // NW
CH = 40
NCH = EPW // CH
NB = 5
NR = NCH // NB
SCH = 2000
NSCH = EPW // SCH
NP_ = 10240
RPT = NP_ // NS

_SC_MESH = plsc.VectorSubcoreMesh(core_axis_name="c", subcore_axis_name="s")


def _zero_vmem_f32(ref, rows, cols):
    zeros16 = jnp.zeros((16,), jnp.float32)

    def zrow(r, _):
        for k in range(cols // 16):
            ref[r, pl.ds(k * 16, 16)] = zeros16
        return 0

    lax.fori_loop(0, rows, zrow, 0)


@functools.partial(
    pl.kernel,
    out_type=jax.ShapeDtypeStruct((NC, NP_, D), jnp.float32),
    mesh=_SC_MESH,
    scratch_types=[
        pltpu.VMEM((2, NB, CH), jnp.int32),
        pltpu.VMEM((2, NB, CH), jnp.int32),
        pltpu.VMEM((NB, CH, D), jnp.float32),
    ]
    + [pltpu.SemaphoreType.DMA] * (3 * NB)
    + [pltpu.VMEM_SHARED((NP_, D), jnp.float32)],
)
def _segsum_sc(src_hbm, dst_hbm, h_hbm, out_hbm, sidx, didx, rows, *rest):
    gsem = rest[0:NB]
    ssem = rest[NB:2 * NB]
    isem = rest[2 * NB:3 * NB]
    acc = rest[3 * NB]

    c = lax.axis_index("c")
    s = lax.axis_index("s")
    wid = s * NC + c
    base = wid * EPW

    _zero_vmem_f32(rows.at[0], CH, D)

    def zblk(b, _):
        pltpu.sync_copy(rows.at[0], acc.at[pl.ds(s * RPT + b * CH, CH)])
        return 0

    lax.fori_loop(0, RPT // CH, zblk, 0)

    for b in range(NB):
        pltpu.sync_copy(src_hbm.at[pl.ds(base + b * CH, CH)], sidx.at[0, b])
        pltpu.sync_copy(dst_hbm.at[pl.ds(base + b * CH, CH)], didx.at[0, b])

    plsc.subcore_barrier()

    for b in range(NB):
        pltpu.async_copy(h_hbm.at[sidx.at[0, b]], rows.at[b], gsem[b])

    def two_rounds(g2, _):
        for p in (0, 1):
            g = 2 * g2 + p
            q = 1 - p

            @pl.when(g < NR - 1)
            def _(g=g, p=p, q=q):
                for b in range(NB):
                    off = base + ((g + 1) * NB + b) * CH
                    pltpu.async_copy(src_hbm.at[pl.ds(off, CH)], sidx.at[q, b], isem[b])
                    pltpu.async_copy(dst_hbm.at[pl.ds(off, CH)], didx.at[q, b], isem[b])

            for b in range(NB):
                pltpu.make_async_copy(h_hbm.at[sidx.at[p, b]], rows.at[b], gsem[b]).wait()
                pltpu.async_copy(rows.at[b], acc.at[didx.at[p, b]], ssem[b], add=True)

            for b in range(NB):
                pltpu.make_async_copy(rows.at[b], acc.at[didx.at[p, b]], ssem[b]).wait()

                @pl.when(g < NR - 1)
                def _(b=b, p=p, q=q):
                    pltpu.make_async_copy(src_hbm.at[pl.ds(base, CH)], sidx.at[q, b], isem[b]).wait()
                    pltpu.make_async_copy(dst_hbm.at[pl.ds(base, CH)], didx.at[q, b], isem[b]).wait()
                    pltpu.async_copy(h_hbm.at[sidx.at[q, b]], rows.at[b], gsem[b])

        return 0

    lax.fori_loop(0, NR // 2, two_rounds, 0)

    plsc.subcore_barrier()

    pltpu.sync_copy(acc.at[pl.ds(s * RPT, RPT)], out_hbm.at[c, pl.ds(s * RPT, RPT)])


@functools.partial(
    pl.kernel,
    out_type=jax.ShapeDtypeStruct((NW, T * NP_), jnp.float32),
    mesh=_SC_MESH,
    compiler_params=pltpu.CompilerParams(needs_layout_passes=False),
    scratch_types=[
        pltpu.VMEM((T * NP_,), jnp.float32),
        pltpu.VMEM((SCH,), jnp.int32),
        pltpu.VMEM((SCH,), jnp.int32),
        pltpu.VMEM((SCH,), jnp.int32),
        pltpu.VMEM((SCH,), jnp.int32),
        pltpu.SemaphoreType.DMA,
        pltpu.SemaphoreType.DMA,
    ],
)
def _counts_sc(dst_hbm, et_hbm, out_hbm, cnt_v, dst_v0, dst_v1, typ_v0, typ_v1,
               isem0, isem1):
    c = lax.axis_index("c")
    s = lax.axis_index("s")
    wid = s * NC + c
    base = wid * EPW
    dbuf = (dst_v0, dst_v1)
    tbuf = (typ_v0, typ_v1)
    isem = (isem0, isem1)

    zeros16 = jnp.zeros((16,), jnp.float32)

    def zrow(r, _):
        for k in range(8):
            cnt_v[pl.ds(r * 128 + k * 16, 16)] = zeros16
        return 0

    lax.fori_loop(0, (T * NP_) // 128, zrow, 0)

    pltpu.sync_copy(dst_hbm.at[pl.ds(base, SCH)], dbuf[0])
    pltpu.sync_copy(et_hbm.at[pl.ds(base, SCH)], tbuf[0])

    ones16 = jnp.ones((16,), jnp.float32)

    for j in range(NSCH):
        p = j % 2
        q = 1 - p
        if j < NSCH - 1:
            off = base + (j + 1) * SCH
            pltpu.async_copy(dst_hbm.at[pl.ds(off, SCH)], dbuf[q], isem[0])
            pltpu.async_copy(et_hbm.at[pl.ds(off, SCH)], tbuf[q], isem[1])

        def grp(k, _, p=p):
            d = dbuf[p][pl.ds(k * 16, 16)]
            t = tbuf[p][pl.ds(k * 16, 16)]
            plsc.addupdate_scatter(cnt_v, [t * NP_ + d], ones16)
            return 0

        lax.fori_loop(0, SCH // 16, grp, 0)

        if j < NSCH - 1:
            pltpu.make_async_copy(dst_hbm.at[pl.ds(base, SCH)], dbuf[q], isem[0]).wait()
            pltpu.make_async_copy(et_hbm.at[pl.ds(base, SCH)], tbuf[q], isem[1]).wait()

        pltpu.sync_copy(cnt_v, out_hbm.at[wid])


R = 1000
GRID = N // R


def _eb_tc_body(c_ref, et_ref, eb_ref):
    cs = jnp.sum(c_ref[...], axis=0)
    eb = jax.lax.dot_general(cs, et_ref[...], ((( 0,), (0,)), ((), ())),
                             preferred_element_type=jnp.float32)
    eb_ref[...] = eb[:N]


_eb_tc = pl.pallas_call(
    _eb_tc_body,
    in_specs=[
        pl.BlockSpec((NW, T, NP_), lambda: (0, 0, 0)),
        pl.BlockSpec((T, D), lambda: (0, 0)),
    ],
    out_specs=pl.BlockSpec((N, D), lambda: (0, 0)),
    out_shape=jax.ShapeDtypeStruct((N, D), jnp.float32),
)


def _mlp_tc_body(x_ref, p_ref, eb_ref, wa_ref, ba_ref, wb_ref, bb_ref, h_ref):
    z = 2.0 * x_ref[...] + p_ref[0] + p_ref[1] + eb_ref[...]
    a = jnp.dot(z, wa_ref[...], preferred_element_type=jnp.float32) + ba_ref[...]
    a = jnp.where(a > 0, a, 0.01 * a)
    h_ref[...] = jnp.dot(a, wb_ref[...], preferred_element_type=jnp.float32) + bb_ref[...]


_mlp_tc = pl.pallas_call(
    _mlp_tc_body,
    grid=(GRID,),
    in_specs=[
        pl.BlockSpec((R, D), lambda i: (i, 0)),
        pl.BlockSpec((NC, R, D), lambda i: (0, i, 0)),
        pl.BlockSpec((R, D), lambda i: (i, 0)),
        pl.BlockSpec((D, D), lambda i: (0, 0)),
        pl.BlockSpec((1, D), lambda i: (0, 0)),
        pl.BlockSpec((D, D), lambda i: (0, 0)),
        pl.BlockSpec((1, D), lambda i: (0, 0)),
    ],
    out_specs=pl.BlockSpec((R, D), lambda i: (i, 0)),
    out_shape=jax.ShapeDtypeStruct((N, D), jnp.float32),
)


def kernel(x, edge_index, edge_type, e_table,
           W1a, b1a, W1b, b1b, W2a, b2a, W2b, b2b, W3a, b3a, W3b, b3b):
    b1a_, b1b_ = b1a.reshape(1, D), b1b.reshape(1, D)
    b2a_, b2b_ = b2a.reshape(1, D), b2b.reshape(1, D)
    b3a_, b3b_ = b3a.reshape(1, D), b3b.reshape(1, D)

    src = edge_index[0]
    dst = edge_index[1]
    C = _counts_sc(dst, edge_type).reshape(NW, T, NP_)
    eb = _eb_tc(C, e_table)
    P0 = _segsum_sc(src, dst, x)
    h1 = _mlp_tc(x, P0, eb, W1a, b1a_, W1b, b1b_)
    P1 = _segsum_sc(src, dst, h1)
    h2 = _mlp_tc(h1, P1, eb, W2a, b2a_, W2b, b2b_)
    P2 = _segsum_sc(src, dst, h2)
    h3 = _mlp_tc(h2, P2, eb, W3a, b3a_, W3b, b3b_)
    return h3
